# baseline (speedup 1.0000x reference)
import jax
import jax.numpy as jnp
from jax import lax
from jax.experimental import pallas as pl
from jax.experimental.pallas import tpu as pltpu
from jax.experimental.pallas import tpu_sc as plsc

N = 1048576
D = 64
DP = 72
NUM_ATOM = 100
NUM_BOND = 10
NC = 2
NS = 16
NW = NC * NS
PER_W = N // NW
CHUNK = 128
NB = 2
G = PER_W // CHUNK
RB = N // 128
L = 16


def _body(atom_ids, bond_ids, atom_table, bond_table, atom_out, bond_out,
          atom_sh, bond_sh, idx_v, rows_v, m_v, isem, gsem, osem):
    cid = lax.axis_index("c")
    sid = lax.axis_index("s")
    wid = sid * NC + cid

    @pl.when(sid == 0)
    def _stage():
        pltpu.sync_copy(atom_table, atom_sh)
        pltpu.sync_copy(bond_table, bond_sh)

    plsc.subcore_barrier()

    base = wid * PER_W
    iota = lax.iota(jnp.int32, L)
    row_idx = [iota + (rrg * L) for rrg in range(CHUNK // L)]

    def do_table(ids_hbm, sh, out4):
        def idx_load(g, b):
            pltpu.make_async_copy(
                ids_hbm.at[pl.ds(base + g * CHUNK, CHUNK)],
                idx_v.at[b], isem.at[b]).start()

        def idx_wait(g, b):
            pltpu.make_async_copy(
                ids_hbm.at[pl.ds(base + g * CHUNK, CHUNK)],
                idx_v.at[b], isem.at[b]).wait()

        def row_gather(b):
            return pltpu.make_async_copy(
                sh.at[idx_v.at[b]], rows_v.at[b], gsem.at[b])

        def out_copy(g, b, c8):
            rbt = (base + g * CHUNK) // 128
            return pltpu.make_async_copy(
                m_v.at[b, pl.ds(c8 * 1024, 1024)], out4.at[c8, rbt],
                osem.at[b])

        def transpose(b):
            @plsc.parallel_loop(0, D, unroll=16)
            def _col(c):
                col = jnp.full((L,), c, jnp.int32)
                for rrg in range(CHUNK // L):
                    vals = plsc.load_gather(
                        rows_v.at[b], [row_idx[rrg], col])
                    m_v[b, pl.ds(c * CHUNK + rrg * L, L)] = vals

        idx_load(0, 0)
        idx_load(1, 1)
        idx_wait(0, 0)
        row_gather(0).start()

        def outer(o, carry):
            for bb in range(NB):
                g = o * NB + bb

                row_gather(bb).wait()

                @pl.when(g + 1 < G)
                def _next_gather():
                    idx_wait(g + 1, 1 - bb)
                    row_gather(1 - bb).start()

                @pl.when(g >= NB)
                def _drain_out():
                    for c8 in range(8):
                        out_copy(g - NB, bb, c8).wait()

                transpose(bb)

                for c8 in range(8):
                    out_copy(g, bb, c8).start()

                @pl.when(g + NB < G)
                def _prefetch():
                    idx_load(g + NB, bb)
            return carry

        lax.fori_loop(0, G // NB, outer, 0)

        for bb in range(NB):
            g = G - NB + bb
            for c8 in range(8):
                out_copy(g, bb, c8).wait()

    do_table(atom_ids, atom_sh, atom_out)
    do_table(bond_ids, bond_sh, bond_out)


@jax.jit
def kernel(atom_ids, bond_ids, atom_table, bond_table):
    mesh = plsc.VectorSubcoreMesh(core_axis_name="c", subcore_axis_name="s")
    out4_t = jax.ShapeDtypeStruct((8, RB, 1024), jnp.float32)
    run = pl.kernel(
        _body,
        out_type=(out4_t, out4_t),
        mesh=mesh,
        scratch_types=[
            pltpu.VMEM_SHARED((NUM_ATOM, DP), jnp.float32),
            pltpu.VMEM_SHARED((NUM_BOND, DP), jnp.float32),
            pltpu.VMEM((NB, CHUNK), jnp.int32),
            pltpu.VMEM((NB, CHUNK, DP), jnp.float32),
            pltpu.VMEM((NB, CHUNK * D), jnp.float32),
            pltpu.SemaphoreType.DMA((NB,)),
            pltpu.SemaphoreType.DMA((NB,)),
            pltpu.SemaphoreType.DMA((NB,)),
        ],
        compiler_params=pltpu.CompilerParams(
            use_tc_tiling_on_sc=False, needs_layout_passes=False),
    )
    atom_pad = jnp.pad(atom_table, ((0, 0), (0, DP - D)))
    bond_pad = jnp.pad(bond_table, ((0, 0), (0, DP - D)))
    a4, b4 = run(atom_ids.astype(jnp.int32), bond_ids.astype(jnp.int32),
                 atom_pad, bond_pad)
    atom_out = a4.reshape(8, RB, 8, 128).transpose(1, 3, 0, 2).reshape(N, D)
    bond_out = b4.reshape(8, RB, 8, 128).transpose(1, 3, 0, 2).reshape(N, D)
    return (atom_out, bond_out)


# speedup vs baseline: 1.6868x; 1.6868x over previous
import jax
import jax.numpy as jnp
from jax import lax
from jax.experimental import pallas as pl
from jax.experimental.pallas import tpu as pltpu
from jax.experimental.pallas import tpu_sc as plsc

N = 1048576
D = 64
DP = 72
NUM_ATOM = 100
NUM_BOND = 10
NC = 2
NS = 16
NW = NC * NS
PER_W = N // NW
CHUNK = 128
K = 4
G = PER_W // CHUNK
GS = G // K
RB = N // 128
L = 16
MW = 8 * K * 128
MSZ = 8 * K * 1024


def _body(atom_ids, bond_ids, atom_table, bond_table, atom_out, bond_out,
          atom_sh, bond_sh, idx_v, rows_v, m_v, isem, gsem, osem):
    cid = lax.axis_index("c")
    sid = lax.axis_index("s")
    wid = sid * NC + cid

    @pl.when(sid == 0)
    def _stage():
        pltpu.sync_copy(atom_table, atom_sh)
        pltpu.sync_copy(bond_table, bond_sh)

    plsc.subcore_barrier()

    base = wid * PER_W
    iota = lax.iota(jnp.int32, L)
    row_idx = [iota + (rrg * L) for rrg in range(CHUNK // L)]

    def do_table(ids_hbm, sh, out4):
        def idx_load(s, sb):
            pltpu.make_async_copy(
                ids_hbm.at[pl.ds(base + s * K * CHUNK, K * CHUNK)],
                idx_v.at[sb], isem.at[sb]).start()

        def idx_wait(s, sb):
            pltpu.make_async_copy(
                ids_hbm.at[pl.ds(base + s * K * CHUNK, K * CHUNK)],
                idx_v.at[sb], isem.at[sb]).wait()

        def row_gather(kk, sb, b):
            return pltpu.make_async_copy(
                sh.at[idx_v.at[sb, pl.ds(kk * CHUNK, CHUNK)]],
                rows_v.at[b], gsem.at[b])

        def out_copy(s, mb, c8):
            sbt = (base + s * K * CHUNK) // (K * 128)
            return pltpu.make_async_copy(
                m_v.at[pl.ds(mb * MSZ + c8 * (K * 1024), K * 1024)],
                out4.at[c8, sbt], osem.at[mb])

        def transpose(mb, kk, b):
            @plsc.parallel_loop(0, D, unroll=8)
            def _col(c):
                col = jnp.full((L,), c, jnp.int32)
                mbase = (mb * MSZ + kk * 1024
                         + (c // 8) * (K * 1024) + (c % 8) * 128)
                for rrg in range(CHUNK // L):
                    vals = plsc.load_gather(
                        rows_v.at[b], [row_idx[rrg], col])
                    m_v[pl.ds(mbase + rrg * L, L)] = vals

        idx_load(0, 0)
        idx_load(1, 1)
        idx_wait(0, 0)
        row_gather(0, 0, 0).start()

        def outer(s, carry):
            sb = lax.rem(s, 2)
            mb = lax.rem(s, 2)

            @pl.when(s >= 2)
            def _drain_out():
                for c8 in range(8):
                    out_copy(s - 2, mb, c8).wait()

            for kk in range(K):
                b = kk % 2

                row_gather(kk, sb, b).wait()

                if kk < K - 1:
                    row_gather(kk + 1, sb, 1 - b).start()
                else:
                    @pl.when(s + 1 < GS)
                    def _next_super_gather():
                        idx_wait(s + 1, 1 - sb)
                        row_gather(0, 1 - sb, 1 - b).start()

                transpose(mb, kk, b)

            for c8 in range(8):
                out_copy(s, mb, c8).start()

            @pl.when(s + 2 < GS)
            def _prefetch_ids():
                idx_load(s + 2, sb)

            return carry

        lax.fori_loop(0, GS, outer, 0)

        for ss in range(2):
            s = GS - 2 + ss
            for c8 in range(8):
                out_copy(s, s % 2, c8).wait()

    do_table(atom_ids, atom_sh, atom_out)
    do_table(bond_ids, bond_sh, bond_out)


@jax.jit
def kernel(atom_ids, bond_ids, atom_table, bond_table):
    mesh = plsc.VectorSubcoreMesh(core_axis_name="c", subcore_axis_name="s")
    out4_t = jax.ShapeDtypeStruct((8, RB // K, K * 1024), jnp.float32)
    run = pl.kernel(
        _body,
        out_type=(out4_t, out4_t),
        mesh=mesh,
        scratch_types=[
            pltpu.VMEM_SHARED((NUM_ATOM, DP), jnp.float32),
            pltpu.VMEM_SHARED((NUM_BOND, DP), jnp.float32),
            pltpu.VMEM((2, K * CHUNK), jnp.int32),
            pltpu.VMEM((2, CHUNK, DP), jnp.float32),
            pltpu.VMEM((2 * MSZ,), jnp.float32),
            pltpu.SemaphoreType.DMA((2,)),
            pltpu.SemaphoreType.DMA((2,)),
            pltpu.SemaphoreType.DMA((2,)),
        ],
        compiler_params=pltpu.CompilerParams(
            use_tc_tiling_on_sc=False, needs_layout_passes=False),
    )
    atom_pad = jnp.pad(atom_table, ((0, 0), (0, DP - D)))
    bond_pad = jnp.pad(bond_table, ((0, 0), (0, DP - D)))
    a4, b4 = run(atom_ids.astype(jnp.int32), bond_ids.astype(jnp.int32),
                 atom_pad, bond_pad)
    atom_out = a4.reshape(8, RB, 8, 128).transpose(1, 3, 0, 2).reshape(N, D)
    bond_out = b4.reshape(8, RB, 8, 128).transpose(1, 3, 0, 2).reshape(N, D)
    return (atom_out, bond_out)


# speedup vs baseline: 1.6880x; 1.0007x over previous
import jax
import jax.numpy as jnp
from jax import lax
from jax.experimental import pallas as pl
from jax.experimental.pallas import tpu as pltpu
from jax.experimental.pallas import tpu_sc as plsc

N = 1048576
D = 64
DP = 72
NUM_ATOM = 100
NUM_BOND = 10
NC = 2
NS = 16
NW = NC * NS
PER_W = N // NW
CHUNK = 128
K = 4
G = PER_W // CHUNK
GS = G // K
RB = N // 128
L = 16
MSZ = 8 * K * 1024


def _body(atom_ids, bond_ids, atom_table, bond_table, atom_out, bond_out,
          atom_sh, bond_sh, idx_v, rows_v, m_v, isem, gsem, osem):
    cid = lax.axis_index("c")
    sid = lax.axis_index("s")
    wid = sid * NC + cid

    @pl.when(sid == 0)
    def _stage():
        pltpu.sync_copy(atom_table, atom_sh)
        pltpu.sync_copy(bond_table, bond_sh)

    plsc.subcore_barrier()

    base = wid * PER_W
    iota = lax.iota(jnp.int32, L)
    row_idx = [iota + (rrg * L) for rrg in range(CHUNK // L)]

    def do_table(ids_hbm, sh, out4):
        def idx_load(s, sb):
            pltpu.make_async_copy(
                ids_hbm.at[pl.ds(base + s * K * CHUNK, K * CHUNK)],
                idx_v.at[sb], isem.at[sb]).start()

        def idx_wait(s, sb):
            pltpu.make_async_copy(
                ids_hbm.at[pl.ds(base + s * K * CHUNK, K * CHUNK)],
                idx_v.at[sb], isem.at[sb]).wait()

        def row_gather(kk, sb, b):
            return pltpu.make_async_copy(
                sh.at[idx_v.at[sb, pl.ds(kk * CHUNK, CHUNK)]],
                rows_v.at[b], gsem.at[b])

        def out_copy(s, mb, c8):
            sbt = (base + s * K * CHUNK) // (K * 128)
            return pltpu.make_async_copy(
                m_v.at[pl.ds(mb * MSZ + c8 * (K * 1024), K * 1024)],
                out4.at[c8, sbt], osem.at[mb])

        def transpose(mb, kk, b):
            @plsc.parallel_loop(0, D, unroll=8)
            def _col(c):
                col = jnp.full((L,), c, jnp.int32)
                mbase = (mb * MSZ + kk * 1024
                         + (c // 8) * (K * 1024) + (c % 8) * 128)
                for rrg in range(CHUNK // L):
                    vals = plsc.load_gather(
                        rows_v.at[b], [row_idx[rrg], col])
                    m_v[pl.ds(mbase + rrg * L, L)] = vals

        idx_load(0, 0)
        idx_load(1, 1)
        idx_wait(0, 0)
        row_gather(0, 0, 0).start()

        def outer(s, carry):
            sb = lax.rem(s, 2)
            mb = lax.rem(s, 2)

            @pl.when(s >= 2)
            def _drain_out():
                for c8 in range(8):
                    out_copy(s - 2, mb, c8).wait()

            for kk in range(K):
                b = kk % 2

                row_gather(kk, sb, b).wait()

                if kk < K - 1:
                    row_gather(kk + 1, sb, 1 - b).start()
                else:
                    @pl.when(s + 1 < GS)
                    def _next_super_gather():
                        idx_wait(s + 1, 1 - sb)
                        row_gather(0, 1 - sb, 1 - b).start()

                transpose(mb, kk, b)

            for c8 in range(8):
                out_copy(s, mb, c8).start()

            @pl.when(s + 2 < GS)
            def _prefetch_ids():
                idx_load(s + 2, sb)

            return carry

        lax.fori_loop(0, GS, outer, 0)

        for ss in range(2):
            s = GS - 2 + ss
            for c8 in range(8):
                out_copy(s, s % 2, c8).wait()

    do_table(atom_ids, atom_sh, atom_out)
    do_table(bond_ids, bond_sh, bond_out)


@jax.jit
def kernel(atom_ids, bond_ids, atom_table, bond_table):
    mesh = plsc.VectorSubcoreMesh(core_axis_name="c", subcore_axis_name="s")
    out4_t = jax.ShapeDtypeStruct((8, RB // K, K * 1024), jnp.float32)
    run = pl.kernel(
        _body,
        out_type=(out4_t, out4_t),
        mesh=mesh,
        scratch_types=[
            pltpu.VMEM_SHARED((NUM_ATOM, DP), jnp.float32),
            pltpu.VMEM_SHARED((NUM_BOND, DP), jnp.float32),
            pltpu.VMEM((2, K * CHUNK), jnp.int32),
            pltpu.VMEM((2, CHUNK, DP), jnp.float32),
            pltpu.VMEM((2 * MSZ,), jnp.float32),
            pltpu.SemaphoreType.DMA((2,)),
            pltpu.SemaphoreType.DMA((2,)),
            pltpu.SemaphoreType.DMA((2,)),
        ],
        compiler_params=pltpu.CompilerParams(
            use_tc_tiling_on_sc=False, needs_layout_passes=False),
    )
    atom_pad = jnp.pad(atom_table, ((0, 0), (0, DP - D)))
    bond_pad = jnp.pad(bond_table, ((0, 0), (0, DP - D)))
    a4, b4 = run(atom_ids.astype(jnp.int32), bond_ids.astype(jnp.int32),
                 atom_pad, bond_pad)
    atom_out = a4.reshape(8, RB, 8, 128).transpose(1, 3, 0, 2).reshape(N, D)
    bond_out = b4.reshape(8, RB, 8, 128).transpose(1, 3, 0, 2).reshape(N, D)
    return (atom_out, bond_out)
